# baseline (speedup 1.0000x reference)
import functools

import jax
import jax.numpy as jnp
from jax.experimental import pallas as pl

DEPTH = 1000
SPLIT = 500
DEPTH_PER_BLOCK = 40


def _one_hot_t_block(idx_ref, out_ref, *, d_base):
    d0 = d_base + pl.program_id(0) * DEPTH_PER_BLOCK
    idx = idx_ref[:].astype(jnp.int32)
    n = idx_ref.shape[1]
    drow = jax.lax.broadcasted_iota(jnp.int32, (DEPTH_PER_BLOCK, n), 0) + d0
    out_ref[:] = (drow == idx).astype(jnp.float32)


def _slab(x_row, d_base, d_rows, n):
    num_blocks = d_rows // DEPTH_PER_BLOCK
    return pl.pallas_call(
        functools.partial(_one_hot_t_block, d_base=d_base),
        grid=(num_blocks,),
        in_specs=[pl.BlockSpec((1, n), lambda i: (0, 0))],
        out_specs=pl.BlockSpec((DEPTH_PER_BLOCK, n), lambda i: (i, 0)),
        out_shape=jax.ShapeDtypeStruct((d_rows, n), jnp.float32),
    )(x_row)


def kernel(x):
    n = x.size
    x_row = jnp.reshape(x, (1, n))
    top = _slab(x_row, 0, SPLIT, n)
    bot = _slab(x_row, SPLIT, DEPTH - SPLIT, n)
    out_t = jnp.concatenate([top, bot], axis=0)
    return jnp.reshape(jnp.transpose(out_t), (1, n, DEPTH))


# speedup vs baseline: 2.0840x; 2.0840x over previous
import jax
import jax.numpy as jnp
from jax.experimental import pallas as pl

DEPTH = 1000
DEPTH_PER_BLOCK = 8


def _one_hot_t_block(idx_ref, out_ref):
    d0 = pl.program_id(0) * DEPTH_PER_BLOCK
    idx = idx_ref[:].astype(jnp.int32)
    n = idx_ref.shape[1]
    drow = jax.lax.broadcasted_iota(jnp.int32, (DEPTH_PER_BLOCK, n), 0) + d0
    out_ref[:] = (drow == idx).astype(jnp.float32)


def kernel(x):
    n = x.size
    x_row = jnp.reshape(x, (1, n))
    num_blocks = DEPTH // DEPTH_PER_BLOCK
    out_t = pl.pallas_call(
        _one_hot_t_block,
        grid=(num_blocks,),
        in_specs=[pl.BlockSpec((1, n), lambda i: (0, 0))],
        out_specs=pl.BlockSpec((DEPTH_PER_BLOCK, n), lambda i: (i, 0)),
        out_shape=jax.ShapeDtypeStruct((DEPTH, n), jnp.float32),
    )(x_row)
    return jnp.reshape(jnp.transpose(out_t), (1, n, DEPTH))


# speedup vs baseline: 2.9509x; 1.4160x over previous
import jax
import jax.numpy as jnp
from jax.experimental import pallas as pl

DEPTH = 1000
DEPTH_PER_BLOCK = 40


def _one_hot_t_block(idx_ref, out_ref):
    d0 = pl.program_id(0) * DEPTH_PER_BLOCK
    idx = idx_ref[:].astype(jnp.int32)
    n = idx_ref.shape[1]
    drow = jax.lax.broadcasted_iota(jnp.int32, (DEPTH_PER_BLOCK, n), 0) + d0
    out_ref[:] = (drow == idx).astype(jnp.float32)


def kernel(x):
    n = x.size
    x_row = jnp.reshape(x, (1, n))
    num_blocks = DEPTH // DEPTH_PER_BLOCK
    out_t = pl.pallas_call(
        _one_hot_t_block,
        grid=(num_blocks,),
        in_specs=[pl.BlockSpec((1, n), lambda i: (0, 0))],
        out_specs=pl.BlockSpec((DEPTH_PER_BLOCK, n), lambda i: (i, 0)),
        out_shape=jax.ShapeDtypeStruct((DEPTH, n), jnp.float32),
    )(x_row)
    return jnp.reshape(jnp.transpose(out_t), (1, n, DEPTH))
